# Initial kernel scaffold; baseline (speedup 1.0000x reference)
import jax
import jax.numpy as jnp
from jax.experimental import pallas as pl


def kernel(root_cubes, meta, grids):
    raise NotImplementedError("write your pallas kernel here")


# speedup vs baseline: 10.1683x; 10.1683x over previous
import jax
import jax.numpy as jnp
from jax.experimental import pallas as pl
from jax.experimental.pallas import tpu as pltpu

X, Y, Z = 128, 128, 64
YZ = Y * Z
K = 10
THR = 0.3
NEG = float("-inf")


def _body(x_ref, out_ref, nm_ref):
    x = x_ref[0]
    col = jax.lax.broadcasted_iota(jnp.int32, (X, YZ), 1)
    zid = col % Z

    fill1 = jnp.full((X, 1), NEG, jnp.float32)
    zp = jnp.concatenate([x[:, 1:], fill1], axis=1)
    zp = jnp.where(zid == Z - 1, NEG, zp)
    zm = jnp.concatenate([fill1, x[:, :-1]], axis=1)
    zm = jnp.where(zid == 0, NEG, zm)
    m = jnp.maximum(x, jnp.maximum(zp, zm))
    fillz = jnp.full((X, Z), NEG, jnp.float32)
    yp = jnp.concatenate([m[:, Z:], fillz], axis=1)
    ym = jnp.concatenate([fillz, m[:, :-Z]], axis=1)
    m = jnp.maximum(m, jnp.maximum(yp, ym))
    fillr = jnp.full((1, YZ), NEG, jnp.float32)
    xp = jnp.concatenate([m[1:, :], fillr], axis=0)
    xm = jnp.concatenate([fillr, m[:-1, :]], axis=0)
    m = jnp.maximum(m, jnp.maximum(xp, xm))

    nm = jnp.where(x == m, x, jnp.float32(0.0))
    nm_ref[...] = nm
    rowmax = jnp.max(nm, axis=1, keepdims=True)

    iota_r = jax.lax.broadcasted_iota(jnp.int32, (X, 1), 0)
    iota_c = jax.lax.broadcasted_iota(jnp.int32, (1, YZ), 1)
    lane = jax.lax.broadcasted_iota(jnp.int32, (1, X), 1)

    def step(k, carry):
        rowmax, vals, flats = carry
        v = jnp.max(rowmax)
        r = jnp.min(jnp.where(rowmax == v, iota_r, X))
        row = nm_ref[pl.ds(r, 1), :]
        c = jnp.min(jnp.where(row == v, iota_c, YZ))
        masked = jnp.where(iota_c == c, jnp.float32(-1.0), row)
        nm_ref[pl.ds(r, 1), :] = masked
        rowmax = jnp.where(iota_r == r, jnp.max(masked), rowmax)
        vals = jnp.where(lane == k, v, vals)
        flats = jnp.where(lane == k, r * YZ + c, flats)
        return rowmax, vals, flats

    init = (rowmax, jnp.zeros((1, X), jnp.float32), jnp.zeros((1, X), jnp.int32))
    _, vals, flats = jax.lax.fori_loop(0, K, step, init)

    ix = (flats // YZ).astype(jnp.float32)
    iy = ((flats % YZ) // Z).astype(jnp.float32)
    iz = (flats % Z).astype(jnp.float32)
    fx = ix / jnp.float32(127.0) * jnp.float32(8000.0) + jnp.float32(-4000.0)
    fy = iy / jnp.float32(127.0) * jnp.float32(8000.0) + jnp.float32(-4000.0)
    fz = iz / jnp.float32(63.0) * jnp.float32(2000.0) + jnp.float32(300.0 - 1000.0)
    flag = (vals > jnp.float32(THR)).astype(jnp.float32) - jnp.float32(1.0)

    out_ref[0] = jnp.concatenate([fx, fy, fz, flag, vals], axis=0)


def kernel(root_cubes, meta, grids):
    b = root_cubes.shape[0]
    x2 = root_cubes.reshape(b, X, YZ)
    res = pl.pallas_call(
        _body,
        grid=(b,),
        in_specs=[pl.BlockSpec((1, X, YZ), lambda i: (i, 0, 0))],
        out_specs=pl.BlockSpec((1, 5, X), lambda i: (i, 0, 0)),
        out_shape=jax.ShapeDtypeStruct((b, 5, X), jnp.float32),
        scratch_shapes=[pltpu.VMEM((X, YZ), jnp.float32)],
    )(x2)
    return res[:, :, :K].transpose(0, 2, 1)
